# baseline (speedup 1.0000x reference)
import functools
import jax
import jax.numpy as jnp
from jax import lax
from jax.experimental import pallas as pl
from jax.experimental.pallas import tpu as pltpu
from jax.experimental.pallas import tpu_sc as plsc

N_LIG = 10000
E_LIG = 320000
N_REC = 1000
E_REC = 16000
D = 64
L = 8

NTILES = 32
CH = 64
TW = 128

F32 = jnp.float32


def _cdiv(a, b):
    return (a + b - 1) // b


def _make_edge_sum(n_pad, e_pad, with_messages):
    e_per = e_pad // NTILES
    chunks = e_per // CH
    rows_per_sub = n_pad // 16
    mesh = plsc.VectorSubcoreMesh(core_axis_name="c", subcore_axis_name="s")

    def _zero_acc_slice(zsrc, acc_sh, s):
        base_r = s * rows_per_sub
        nfull = rows_per_sub // CH
        for k in range(nfull):
            pltpu.sync_copy(zsrc, acc_sh.at[pl.ds(base_r + k * CH, CH)])
        rem = rows_per_sub - nfull * CH
        if rem:
            pltpu.sync_copy(zsrc.at[pl.ds(0, rem)],
                            acc_sh.at[pl.ds(base_r + nfull * CH, rem)])
        return base_r

    def _writeback(acc_sh, out_hbm, c, base_r):
        pltpu.sync_copy(acc_sh.at[pl.ds(base_r, rows_per_sub)],
                        out_hbm.at[c, pl.ds(base_r, rows_per_sub)])

    if with_messages:
        def body(t_hbm, src_hbm, dst_hbm, zeros_hbm, out_hbm,
                 src_v, dst_v, bufs_v, bufd_v, mbuf_v, acc_sh, sem_s, sem_d):
            c = lax.axis_index("c")
            s = lax.axis_index("s")
            wid = c * 16 + s
            pltpu.sync_copy(zeros_hbm, mbuf_v)
            base_r = _zero_acc_slice(mbuf_v, acc_sh, s)
            plsc.subcore_barrier()

            ebase = wid * e_per

            @pl.loop(0, chunks)
            def _chunk(i):
                off = ebase + i * CH
                pltpu.sync_copy(dst_hbm.at[pl.ds(off, CH)], dst_v)
                pltpu.sync_copy(src_hbm.at[pl.ds(off, CH)], src_v)
                cp_s = pltpu.async_copy(t_hbm.at[src_v], bufs_v, sem_s)
                cp_d = pltpu.async_copy(t_hbm.at[dst_v], bufd_v, sem_d)
                cp_s.wait()
                cp_d.wait()

                @pl.loop(0, CH)
                def _row(r):
                    for j in range(D // 16):
                        x = (bufs_v[r, pl.ds(j * 16, 16)] +
                             bufd_v[r, pl.ds(D + j * 16, 16)])
                        mbuf_v[r, pl.ds(j * 16, 16)] = jnp.maximum(x, 0.01 * x)

                pltpu.sync_copy(mbuf_v, acc_sh.at[dst_v], add=True)

            plsc.subcore_barrier()
            _writeback(acc_sh, out_hbm, c, base_r)

        scratch = [
            pltpu.VMEM((CH,), jnp.int32),
            pltpu.VMEM((CH,), jnp.int32),
            pltpu.VMEM((CH, TW), F32),
            pltpu.VMEM((CH, TW), F32),
            pltpu.VMEM((CH, TW), F32),
            pltpu.VMEM_SHARED((n_pad, TW), F32),
            pltpu.SemaphoreType.DMA,
            pltpu.SemaphoreType.DMA,
        ]
    else:
        def body(dst_hbm, ones_hbm, out_hbm, dst_v, ones_v, zbuf_v, acc_sh):
            c = lax.axis_index("c")
            s = lax.axis_index("s")
            wid = c * 16 + s
            pltpu.sync_copy(ones_hbm, ones_v)

            @pl.loop(0, CH)
            def _zero_rows(r):
                for j in range(TW // 16):
                    zbuf_v[r, pl.ds(j * 16, 16)] = jnp.zeros((16,), F32)

            base_r = _zero_acc_slice(zbuf_v, acc_sh, s)
            plsc.subcore_barrier()

            ebase = wid * e_per

            @pl.loop(0, chunks)
            def _chunk(i):
                off = ebase + i * CH
                pltpu.sync_copy(dst_hbm.at[pl.ds(off, CH)], dst_v)
                pltpu.sync_copy(ones_v, acc_sh.at[dst_v], add=True)

            plsc.subcore_barrier()
            _writeback(acc_sh, out_hbm, c, base_r)

        scratch = [
            pltpu.VMEM((CH,), jnp.int32),
            pltpu.VMEM((CH, TW), F32),
            pltpu.VMEM((CH, TW), F32),
            pltpu.VMEM_SHARED((n_pad, TW), F32),
        ]

    return pl.kernel(
        body,
        out_type=jax.ShapeDtypeStruct((2, n_pad, TW), F32),
        mesh=mesh,
        scratch_types=scratch,
        name="edge_msgsum" if with_messages else "edge_degree",
    )


_BLK = 512


def _proj_body(h_ref, w_ref, b_ref, t_ref, q_ref, k_ref):
    y = jnp.dot(h_ref[...], w_ref[...], preferred_element_type=F32) + b_ref[...]
    t_ref[...] = y[:, 0:2 * D]
    q_ref[...] = y[:, 2 * D:3 * D]
    k_ref[...] = y[:, 3 * D:4 * D]


def _proj(h, wc, bc, n_pad):
    grid = (_cdiv(n_pad, _BLK),)
    return pl.pallas_call(
        _proj_body,
        grid=grid,
        in_specs=[
            pl.BlockSpec((_BLK, D), lambda i: (i, 0)),
            pl.BlockSpec((D, 4 * D), lambda i: (0, 0)),
            pl.BlockSpec((1, 4 * D), lambda i: (0, 0)),
        ],
        out_specs=[
            pl.BlockSpec((_BLK, 2 * D), lambda i: (i, 0)),
            pl.BlockSpec((_BLK, D), lambda i: (i, 0)),
            pl.BlockSpec((_BLK, D), lambda i: (i, 0)),
        ],
        out_shape=[
            jax.ShapeDtypeStruct((n_pad, 2 * D), F32),
            jax.ShapeDtypeStruct((n_pad, D), F32),
            jax.ShapeDtypeStruct((n_pad, D), F32),
        ],
    )(h, wc, bc.reshape(1, 4 * D))


def _attn_small_body(q_ref, k_ref, v_ref, o_ref):
    s = lax.dot_general(q_ref[...], k_ref[...], (((1,), (1,)), ((), ())),
                        preferred_element_type=F32) * (1.0 / 8.0)
    s = s - jnp.max(s, axis=-1, keepdims=True)
    p = jnp.exp(s)
    p = p / jnp.sum(p, axis=-1, keepdims=True)
    o_ref[...] = jnp.dot(p, v_ref[...], preferred_element_type=F32)


def _attn_small(q, k, v, n_out):
    nk = k.shape[0]
    grid = (_cdiv(n_out, _BLK),)
    return pl.pallas_call(
        _attn_small_body,
        grid=grid,
        in_specs=[
            pl.BlockSpec((_BLK, D), lambda i: (i, 0)),
            pl.BlockSpec((nk, D), lambda i: (0, 0)),
            pl.BlockSpec((nk, D), lambda i: (0, 0)),
        ],
        out_specs=pl.BlockSpec((_BLK, D), lambda i: (i, 0)),
        out_shape=jax.ShapeDtypeStruct((n_out, D), F32),
    )(q, k, v)


def _attn_flash_body(q_ref, k_ref, v_ref, o_ref, m_ref, l_ref, acc_ref):
    i = pl.program_id(0)

    @pl.when(i == 0)
    def _():
        m_ref[...] = jnp.full_like(m_ref, -jnp.inf)
        l_ref[...] = jnp.zeros_like(l_ref)
        acc_ref[...] = jnp.zeros_like(acc_ref)

    s = lax.dot_general(q_ref[...], k_ref[...], (((1,), (1,)), ((), ())),
                        preferred_element_type=F32) * (1.0 / 8.0)
    m_prev = m_ref[...]
    m_cur = jnp.maximum(m_prev, jnp.max(s, axis=-1, keepdims=True))
    alpha = jnp.exp(m_prev - m_cur)
    p = jnp.exp(s - m_cur)
    l_ref[...] = l_ref[...] * alpha + jnp.sum(p, axis=-1, keepdims=True)
    acc_ref[...] = acc_ref[...] * alpha + jnp.dot(p, v_ref[...],
                                                  preferred_element_type=F32)
    m_ref[...] = m_cur

    @pl.when(i == pl.num_programs(0) - 1)
    def _():
        o_ref[...] = acc_ref[...] / l_ref[...]


def _attn_flash(q, k, v, kblk):
    nq = q.shape[0]
    nk_total = v.shape[0]
    grid = (nk_total // kblk,)
    return pl.pallas_call(
        _attn_flash_body,
        grid=grid,
        in_specs=[
            pl.BlockSpec((nq, D), lambda i: (0, 0)),
            pl.BlockSpec((kblk, D), lambda i: (i, 0)),
            pl.BlockSpec((kblk, D), lambda i: (i, 0)),
        ],
        out_specs=pl.BlockSpec((nq, D), lambda i: (0, 0)),
        out_shape=jax.ShapeDtypeStruct((nq, D), F32),
        scratch_shapes=[
            pltpu.VMEM((nq, 1), F32),
            pltpu.VMEM((nq, 1), F32),
            pltpu.VMEM((nq, D), F32),
        ],
    )(q, k, v)


def _update_body(h_ref, s0_ref, s1_ref, d0_ref, d1_ref, cr_ref, u_ref, b_ref,
                 o_ref):
    agg = ((s0_ref[:, :D] + s1_ref[:, :D]) /
           jnp.maximum(d0_ref[:, :D] + d1_ref[:, :D], 1.0))
    x = jnp.concatenate([h_ref[...], agg, cr_ref[...]], axis=-1)
    z = jnp.dot(x, u_ref[...], preferred_element_type=F32) + b_ref[...]
    o_ref[...] = h_ref[...] + jnp.maximum(z, 0.01 * z)


def _update(h, s0, s1, d0, d1, cr, u, b):
    n = h.shape[0]
    grid = (_cdiv(n, _BLK),)
    blk = pl.BlockSpec((_BLK, D), lambda i: (i, 0))
    wblk = pl.BlockSpec((_BLK, TW), lambda i: (i, 0))
    return pl.pallas_call(
        _update_body,
        grid=grid,
        in_specs=[blk, wblk, wblk, wblk, wblk, blk,
                  pl.BlockSpec((3 * D, D), lambda i: (0, 0)),
                  pl.BlockSpec((1, D), lambda i: (0, 0))],
        out_specs=blk,
        out_shape=jax.ShapeDtypeStruct((n, D), F32),
    )(h, s0, s1, d0, d1, cr, u, b.reshape(1, D))


def _mlp_body(h_ref, w1_ref, b1_ref, w2_ref, b2_ref, w3_ref, b3_ref, o_ref):
    x = jnp.dot(h_ref[...], w1_ref[...], preferred_element_type=F32) + b1_ref[...]
    x = jnp.maximum(x, 0.2 * x)
    x = jnp.dot(x, w2_ref[...], preferred_element_type=F32) + b2_ref[...]
    x = jnp.maximum(x, 0.2 * x)
    o_ref[...] = jnp.dot(x, w3_ref[...], preferred_element_type=F32) + b3_ref[...]


def _mlp(h, w1, b1, w2, b2, w3, b3):
    n = h.shape[0]
    grid = (_cdiv(n, _BLK),)
    return pl.pallas_call(
        _mlp_body,
        grid=grid,
        in_specs=[
            pl.BlockSpec((_BLK, D), lambda i: (i, 0)),
            pl.BlockSpec((D, D), lambda i: (0, 0)),
            pl.BlockSpec((1, D), lambda i: (0, 0)),
            pl.BlockSpec((D, 16), lambda i: (0, 0)),
            pl.BlockSpec((1, 16), lambda i: (0, 0)),
            pl.BlockSpec((16, 1), lambda i: (0, 0)),
            pl.BlockSpec((1, 1), lambda i: (0, 0)),
        ],
        out_specs=pl.BlockSpec((_BLK, 1), lambda i: (i, 0)),
        out_shape=jax.ShapeDtypeStruct((n, 1), F32),
    )(h, w1, b1.reshape(1, D), w2, b2.reshape(1, 16), w3, b3.reshape(1, 1))


def _pad_edges(edge_index, n, e_pad):
    src = edge_index[0].astype(jnp.int32)
    dst = edge_index[1].astype(jnp.int32)
    e = src.shape[0]
    pad = e_pad - e
    src = jnp.pad(src, (0, pad))
    dst = jnp.pad(dst, (0, pad), constant_values=n)
    return src, dst


def kernel(lig_x, lig_edge_index, rec_x, rec_edge_index, W_msg, b_msg, W_q,
           W_k, W_upd, b_upd, out_W1, out_b1, out_W2, out_b2, out_W3, out_b3):
    n_pad_l = 10112
    n_pad_r = 1024
    unit = NTILES * CH * 4
    e_pad_l = _cdiv(E_LIG, unit) * unit
    e_pad_r = _cdiv(E_REC, unit) * unit

    src_l, dst_l = _pad_edges(lig_edge_index, N_LIG, e_pad_l)
    src_r, dst_r = _pad_edges(rec_edge_index, N_REC, e_pad_r)

    zeros_blk = jnp.zeros((CH, TW), F32)
    ones_blk = jnp.ones((CH, TW), F32)

    deg_l_k = _make_edge_sum(n_pad_l, e_pad_l, with_messages=False)
    deg_r_k = _make_edge_sum(n_pad_r, e_pad_r, with_messages=False)
    msg_l_k = _make_edge_sum(n_pad_l, e_pad_l, with_messages=True)
    msg_r_k = _make_edge_sum(n_pad_r, e_pad_r, with_messages=True)

    deg_l = deg_l_k(dst_l, ones_blk)
    deg_r = deg_r_k(dst_r, ones_blk)

    wc = jnp.concatenate([W_msg[:, :D, :], W_msg[:, D:, :], W_q, W_k], axis=2)
    bc = jnp.concatenate(
        [jnp.zeros((L, D), F32), b_msg, jnp.zeros((L, 2 * D), F32)], axis=1)

    h_l, h_r = lig_x, rec_x
    for i in range(L):
        t_l, q_l, k_l = _proj(h_l, wc[i], bc[i], n_pad_l)
        t_r, q_r, k_r = _proj(h_r, wc[i], bc[i], n_pad_r)

        sum_l = msg_l_k(t_l, src_l, dst_l, zeros_blk)
        sum_r = msg_r_k(t_r, src_r, dst_r, zeros_blk)

        cr_l = _attn_small(q_l[:N_LIG], k_r[:N_REC], h_r, N_LIG)
        cr_r = _attn_flash(q_r[:N_REC], k_l[:N_LIG], h_l, kblk=2000)

        h_l = _update(h_l, sum_l[0], sum_l[1], deg_l[0], deg_l[1], cr_l,
                      W_upd[i], b_upd[i])
        h_r = _update(h_r, sum_r[0], sum_r[1], deg_r[0], deg_r[1], cr_r,
                      W_upd[i], b_upd[i])

    return _mlp(h_l, out_W1, out_b1, out_W2, out_b2, out_W3, out_b3)


# speedup vs baseline: 1.5125x; 1.5125x over previous
import functools
import jax
import jax.numpy as jnp
from jax import lax
from jax.experimental import pallas as pl
from jax.experimental.pallas import tpu as pltpu
from jax.experimental.pallas import tpu_sc as plsc

N_LIG = 10000
E_LIG = 320000
N_REC = 1000
E_REC = 16000
D = 64
L = 8

NTILES = 32
CH = 64
TW = 128

F32 = jnp.float32


def _cdiv(a, b):
    return (a + b - 1) // b


def _make_edge_sum(n_pad, e_pad, with_messages):
    e_per = e_pad // NTILES
    chunks = e_per // CH
    rows_per_sub = n_pad // 16
    mesh = plsc.VectorSubcoreMesh(core_axis_name="c", subcore_axis_name="s")

    def _zero_acc_slice(zsrc, acc_sh, s):
        base_r = s * rows_per_sub
        nfull = rows_per_sub // CH
        for k in range(nfull):
            pltpu.sync_copy(zsrc, acc_sh.at[pl.ds(base_r + k * CH, CH)])
        rem = rows_per_sub - nfull * CH
        if rem:
            pltpu.sync_copy(zsrc.at[pl.ds(0, rem)],
                            acc_sh.at[pl.ds(base_r + nfull * CH, rem)])
        return base_r

    def _writeback(acc_sh, out_hbm, c, base_r):
        pltpu.sync_copy(acc_sh.at[pl.ds(base_r, rows_per_sub)],
                        out_hbm.at[c, pl.ds(base_r, rows_per_sub)])

    if with_messages:
        def body(t_hbm, src_hbm, dst_hbm, zeros_hbm, out_hbm,
                 src_v, dst_v, bufs_v, bufd_v, mbuf_v, acc_sh, sem_s, sem_d):
            c = lax.axis_index("c")
            s = lax.axis_index("s")
            wid = c * 16 + s
            pltpu.sync_copy(zeros_hbm, mbuf_v)
            base_r = _zero_acc_slice(mbuf_v, acc_sh, s)
            plsc.subcore_barrier()

            ebase = wid * e_per

            @pl.loop(0, chunks)
            def _chunk(i):
                off = ebase + i * CH
                pltpu.sync_copy(dst_hbm.at[pl.ds(off, CH)], dst_v)
                pltpu.sync_copy(src_hbm.at[pl.ds(off, CH)], src_v)
                cp_s = pltpu.async_copy(t_hbm.at[src_v], bufs_v, sem_s)
                cp_d = pltpu.async_copy(t_hbm.at[dst_v], bufd_v, sem_d)
                cp_s.wait()
                cp_d.wait()

                @pl.loop(0, CH)
                def _row(r):
                    for j in range(D // 16):
                        x = (bufs_v[r, pl.ds(j * 16, 16)] +
                             bufd_v[r, pl.ds(D + j * 16, 16)])
                        mbuf_v[r, pl.ds(j * 16, 16)] = jnp.maximum(x, 0.01 * x)

                pltpu.sync_copy(mbuf_v, acc_sh.at[dst_v], add=True)

            plsc.subcore_barrier()
            _writeback(acc_sh, out_hbm, c, base_r)

        scratch = [
            pltpu.VMEM((CH,), jnp.int32),
            pltpu.VMEM((CH,), jnp.int32),
            pltpu.VMEM((CH, TW), F32),
            pltpu.VMEM((CH, TW), F32),
            pltpu.VMEM((CH, TW), F32),
            pltpu.VMEM_SHARED((n_pad, TW), F32),
            pltpu.SemaphoreType.DMA,
            pltpu.SemaphoreType.DMA,
        ]
    else:
        def body(dst_hbm, ones_hbm, out_hbm, dst_v, ones_v, zbuf_v, acc_sh):
            c = lax.axis_index("c")
            s = lax.axis_index("s")
            wid = c * 16 + s
            pltpu.sync_copy(ones_hbm, ones_v)

            @pl.loop(0, CH)
            def _zero_rows(r):
                for j in range(TW // 16):
                    zbuf_v[r, pl.ds(j * 16, 16)] = jnp.zeros((16,), F32)

            base_r = _zero_acc_slice(zbuf_v, acc_sh, s)
            plsc.subcore_barrier()

            ebase = wid * e_per

            @pl.loop(0, chunks)
            def _chunk(i):
                off = ebase + i * CH
                pltpu.sync_copy(dst_hbm.at[pl.ds(off, CH)], dst_v)
                pltpu.sync_copy(ones_v, acc_sh.at[dst_v], add=True)

            plsc.subcore_barrier()
            _writeback(acc_sh, out_hbm, c, base_r)

        scratch = [
            pltpu.VMEM((CH,), jnp.int32),
            pltpu.VMEM((CH, TW), F32),
            pltpu.VMEM((CH, TW), F32),
            pltpu.VMEM_SHARED((n_pad, TW), F32),
        ]

    return pl.kernel(
        body,
        out_type=jax.ShapeDtypeStruct((2, n_pad, TW), F32),
        mesh=mesh,
        scratch_types=scratch,
        name="edge_msgsum" if with_messages else "edge_degree",
    )


_BLK = 512


def _proj_body(h_ref, w_ref, b_ref, t_ref, q_ref, k_ref):
    y = jnp.dot(h_ref[...], w_ref[...], preferred_element_type=F32) + b_ref[...]
    t_ref[...] = y[:, 0:2 * D]
    q_ref[...] = y[:, 2 * D:3 * D]
    k_ref[...] = y[:, 3 * D:4 * D]


def _proj(h, wc, bc, n_pad):
    grid = (_cdiv(n_pad, _BLK),)
    return pl.pallas_call(
        _proj_body,
        grid=grid,
        in_specs=[
            pl.BlockSpec((_BLK, D), lambda i: (i, 0)),
            pl.BlockSpec((D, 4 * D), lambda i: (0, 0)),
            pl.BlockSpec((1, 4 * D), lambda i: (0, 0)),
        ],
        out_specs=[
            pl.BlockSpec((_BLK, 2 * D), lambda i: (i, 0)),
            pl.BlockSpec((_BLK, D), lambda i: (i, 0)),
            pl.BlockSpec((_BLK, D), lambda i: (i, 0)),
        ],
        out_shape=[
            jax.ShapeDtypeStruct((n_pad, 2 * D), F32),
            jax.ShapeDtypeStruct((n_pad, D), F32),
            jax.ShapeDtypeStruct((n_pad, D), F32),
        ],
    )(h, wc, bc.reshape(1, 4 * D))


def _attn_small_body(q_ref, k_ref, v_ref, o_ref):
    s = lax.dot_general(q_ref[...], k_ref[...], (((1,), (1,)), ((), ())),
                        preferred_element_type=F32) * (1.0 / 8.0)
    s = s - jnp.max(s, axis=-1, keepdims=True)
    p = jnp.exp(s)
    p = p / jnp.sum(p, axis=-1, keepdims=True)
    o_ref[...] = jnp.dot(p, v_ref[...], preferred_element_type=F32)


def _attn_small(q, k, v, n_out):
    nk = k.shape[0]
    grid = (_cdiv(n_out, _BLK),)
    return pl.pallas_call(
        _attn_small_body,
        grid=grid,
        in_specs=[
            pl.BlockSpec((_BLK, D), lambda i: (i, 0)),
            pl.BlockSpec((nk, D), lambda i: (0, 0)),
            pl.BlockSpec((nk, D), lambda i: (0, 0)),
        ],
        out_specs=pl.BlockSpec((_BLK, D), lambda i: (i, 0)),
        out_shape=jax.ShapeDtypeStruct((n_out, D), F32),
    )(q, k, v)


def _attn_flash_body(q_ref, k_ref, v_ref, o_ref, m_ref, l_ref, acc_ref):
    i = pl.program_id(0)

    @pl.when(i == 0)
    def _():
        m_ref[...] = jnp.full_like(m_ref, -jnp.inf)
        l_ref[...] = jnp.zeros_like(l_ref)
        acc_ref[...] = jnp.zeros_like(acc_ref)

    s = lax.dot_general(q_ref[...], k_ref[...], (((1,), (1,)), ((), ())),
                        preferred_element_type=F32) * (1.0 / 8.0)
    m_prev = m_ref[...]
    m_cur = jnp.maximum(m_prev, jnp.max(s, axis=-1, keepdims=True))
    alpha = jnp.exp(m_prev - m_cur)
    p = jnp.exp(s - m_cur)
    l_ref[...] = l_ref[...] * alpha + jnp.sum(p, axis=-1, keepdims=True)
    acc_ref[...] = acc_ref[...] * alpha + jnp.dot(p, v_ref[...],
                                                  preferred_element_type=F32)
    m_ref[...] = m_cur

    @pl.when(i == pl.num_programs(0) - 1)
    def _():
        o_ref[...] = acc_ref[...] / l_ref[...]


def _attn_flash(q, k, v, kblk):
    nq = q.shape[0]
    nk_total = v.shape[0]
    grid = (nk_total // kblk,)
    return pl.pallas_call(
        _attn_flash_body,
        grid=grid,
        in_specs=[
            pl.BlockSpec((nq, D), lambda i: (0, 0)),
            pl.BlockSpec((kblk, D), lambda i: (i, 0)),
            pl.BlockSpec((kblk, D), lambda i: (i, 0)),
        ],
        out_specs=pl.BlockSpec((nq, D), lambda i: (0, 0)),
        out_shape=jax.ShapeDtypeStruct((nq, D), F32),
        scratch_shapes=[
            pltpu.VMEM((nq, 1), F32),
            pltpu.VMEM((nq, 1), F32),
            pltpu.VMEM((nq, D), F32),
        ],
    )(q, k, v)


def _update_body(h_ref, s0_ref, s1_ref, d0_ref, d1_ref, cr_ref, u_ref, b_ref,
                 o_ref):
    agg = ((s0_ref[:, :D] + s1_ref[:, :D]) /
           jnp.maximum(d0_ref[:, :D] + d1_ref[:, :D], 1.0))
    x = jnp.concatenate([h_ref[...], agg, cr_ref[...]], axis=-1)
    z = jnp.dot(x, u_ref[...], preferred_element_type=F32) + b_ref[...]
    o_ref[...] = h_ref[...] + jnp.maximum(z, 0.01 * z)


def _update(h, s0, s1, d0, d1, cr, u, b):
    n = h.shape[0]
    grid = (_cdiv(n, _BLK),)
    blk = pl.BlockSpec((_BLK, D), lambda i: (i, 0))
    wblk = pl.BlockSpec((_BLK, TW), lambda i: (i, 0))
    return pl.pallas_call(
        _update_body,
        grid=grid,
        in_specs=[blk, wblk, wblk, wblk, wblk, blk,
                  pl.BlockSpec((3 * D, D), lambda i: (0, 0)),
                  pl.BlockSpec((1, D), lambda i: (0, 0))],
        out_specs=blk,
        out_shape=jax.ShapeDtypeStruct((n, D), F32),
    )(h, s0, s1, d0, d1, cr, u, b.reshape(1, D))


def _mlp_body(h_ref, w1_ref, b1_ref, w2_ref, b2_ref, w3_ref, b3_ref, o_ref):
    x = jnp.dot(h_ref[...], w1_ref[...], preferred_element_type=F32) + b1_ref[...]
    x = jnp.maximum(x, 0.2 * x)
    x = jnp.dot(x, w2_ref[...], preferred_element_type=F32) + b2_ref[...]
    x = jnp.maximum(x, 0.2 * x)
    o_ref[...] = jnp.dot(x, w3_ref[...], preferred_element_type=F32) + b3_ref[...]


def _mlp(h, w1, b1, w2, b2, w3, b3):
    n = h.shape[0]
    grid = (_cdiv(n, _BLK),)
    return pl.pallas_call(
        _mlp_body,
        grid=grid,
        in_specs=[
            pl.BlockSpec((_BLK, D), lambda i: (i, 0)),
            pl.BlockSpec((D, D), lambda i: (0, 0)),
            pl.BlockSpec((1, D), lambda i: (0, 0)),
            pl.BlockSpec((D, 16), lambda i: (0, 0)),
            pl.BlockSpec((1, 16), lambda i: (0, 0)),
            pl.BlockSpec((16, 1), lambda i: (0, 0)),
            pl.BlockSpec((1, 1), lambda i: (0, 0)),
        ],
        out_specs=pl.BlockSpec((_BLK, 1), lambda i: (i, 0)),
        out_shape=jax.ShapeDtypeStruct((n, 1), F32),
    )(h, w1, b1.reshape(1, D), w2, b2.reshape(1, 16), w3, b3.reshape(1, 1))


def _pad_edges(edge_index, n, n_pad, e_pad):
    src = edge_index[0].astype(jnp.int32)
    dst = edge_index[1].astype(jnp.int32)
    e = src.shape[0]
    pad = e_pad - e
    src = jnp.pad(src, (0, pad))
    trash = n + jnp.arange(pad, dtype=jnp.int32) % (n_pad - n)
    dst = jnp.concatenate([dst, trash])
    return src, dst


def kernel(lig_x, lig_edge_index, rec_x, rec_edge_index, W_msg, b_msg, W_q,
           W_k, W_upd, b_upd, out_W1, out_b1, out_W2, out_b2, out_W3, out_b3):
    n_pad_l = 10112
    n_pad_r = 1024
    unit = NTILES * CH
    e_pad_l = _cdiv(E_LIG, unit) * unit
    e_pad_r = _cdiv(E_REC, unit) * unit

    src_l, dst_l = _pad_edges(lig_edge_index, N_LIG, n_pad_l, e_pad_l)
    src_r, dst_r = _pad_edges(rec_edge_index, N_REC, n_pad_r, e_pad_r)

    zeros_blk = jnp.zeros((CH, TW), F32)
    ones_blk = jnp.ones((CH, TW), F32)

    deg_l_k = _make_edge_sum(n_pad_l, e_pad_l, with_messages=False)
    deg_r_k = _make_edge_sum(n_pad_r, e_pad_r, with_messages=False)
    msg_l_k = _make_edge_sum(n_pad_l, e_pad_l, with_messages=True)
    msg_r_k = _make_edge_sum(n_pad_r, e_pad_r, with_messages=True)

    deg_l = deg_l_k(dst_l, ones_blk)
    deg_r = deg_r_k(dst_r, ones_blk)

    wc = jnp.concatenate([W_msg[:, :D, :], W_msg[:, D:, :], W_q, W_k], axis=2)
    bc = jnp.concatenate(
        [jnp.zeros((L, D), F32), b_msg, jnp.zeros((L, 2 * D), F32)], axis=1)

    h_l, h_r = lig_x, rec_x
    for i in range(L):
        t_l, q_l, k_l = _proj(h_l, wc[i], bc[i], n_pad_l)
        t_r, q_r, k_r = _proj(h_r, wc[i], bc[i], n_pad_r)

        sum_l = msg_l_k(t_l, src_l, dst_l, zeros_blk)
        sum_r = msg_r_k(t_r, src_r, dst_r, zeros_blk)

        cr_l = _attn_small(q_l[:N_LIG], k_r[:N_REC], h_r, N_LIG)
        cr_r = _attn_flash(q_r[:N_REC], k_l[:N_LIG], h_l, kblk=2000)

        h_l = _update(h_l, sum_l[0], sum_l[1], deg_l[0], deg_l[1], cr_l,
                      W_upd[i], b_upd[i])
        h_r = _update(h_r, sum_r[0], sum_r[1], deg_r[0], deg_r[1], cr_r,
                      W_upd[i], b_upd[i])

    return _mlp(h_l, out_W1, out_b1, out_W2, out_b2, out_W3, out_b3)
